# baseline (speedup 1.0000x reference)
import jax
import jax.numpy as jnp
from jax import lax
from jax.experimental import pallas as pl
from jax.experimental.pallas import tpu as pltpu
from jax.experimental.pallas import tpu_sc as plsc

BATCH = 4
SEQ_LEN = 8192
EMB = 1024
N = BATCH * SEQ_LEN
NUM_CORES = 2
NUM_SUBCORES = 16
NW = NUM_CORES * NUM_SUBCORES
PER_W = N // NW
CHUNK = 16
NCHUNK = PER_W // CHUNK
NBUF = 4
SNBUF = 2


def _gather_body(idx_hbm, table_hbm, out_hbm, idx_v, spbuf, *rest):
    tbufs = rest[:NBUF]
    gsems = rest[NBUF:2 * NBUF]
    psems = rest[2 * NBUF:3 * NBUF]
    wsems = rest[3 * NBUF:]
    cid = lax.axis_index("c")
    sid = lax.axis_index("s")
    wid = sid * NUM_CORES + cid
    base = wid * PER_W
    pltpu.sync_copy(idx_hbm.at[wid], idx_v)
    gcp = [None] * NBUF
    wcp = [None] * SNBUF
    for p in range(2):
        gcp[p] = pltpu.async_copy(table_hbm.at[idx_v.at[p]], tbufs[p], gsems[p])
    for c in range(NCHUNK):
        b = c % NBUF
        s = c % SNBUF
        gcp[b].wait()
        if c >= SNBUF:
            wcp[s].wait()
        pltpu.async_copy(tbufs[b], spbuf.at[sid, s], psems[b]).wait()
        wcp[s] = pltpu.async_copy(
            spbuf.at[sid, s], out_hbm.at[pl.ds(base + c * CHUNK, CHUNK)],
            wsems[s],
        )
        g = c + 2
        if g < NCHUNK:
            gb = g % NBUF
            gcp[gb] = pltpu.async_copy(
                table_hbm.at[idx_v.at[g]], tbufs[gb], gsems[gb]
            )
    for s in range(SNBUF):
        wcp[s].wait()


@jax.jit
def kernel(position_ids, embeddings_table):
    idx = position_ids.reshape(NW, NCHUNK, CHUNK)
    out = pl.kernel(
        _gather_body,
        out_type=jax.ShapeDtypeStruct((N, EMB), jnp.float32),
        mesh=plsc.VectorSubcoreMesh(core_axis_name="c", subcore_axis_name="s"),
        scratch_types=(
            [
                pltpu.VMEM((NCHUNK, CHUNK), jnp.int32),
                pltpu.VMEM_SHARED((NUM_SUBCORES, SNBUF, CHUNK, EMB), jnp.float32),
            ]
            + [pltpu.VMEM((CHUNK, EMB), jnp.float32)] * NBUF
            + [pltpu.SemaphoreType.DMA] * (3 * NBUF)
        ),
    )(idx, embeddings_table)
    return out.reshape(BATCH, SEQ_LEN, EMB)


# speedup vs baseline: 1.0171x; 1.0171x over previous
import jax
import jax.numpy as jnp
from jax import lax
from jax.experimental import pallas as pl
from jax.experimental.pallas import tpu as pltpu
from jax.experimental.pallas import tpu_sc as plsc

BATCH = 4
SEQ_LEN = 8192
EMB = 1024
N = BATCH * SEQ_LEN
NUM_CORES = 2
NUM_SUBCORES = 16
NW = NUM_CORES * NUM_SUBCORES
PER_W = N // NW
CHUNK = 32
NCHUNK = PER_W // CHUNK
NBUF = 3


def _gather_body(idx_hbm, table_hbm, out_hbm, idx_v, *rest):
    bufs = rest[:NBUF]
    gsems = rest[NBUF:2 * NBUF]
    wsems = rest[2 * NBUF:]
    cid = lax.axis_index("c")
    sid = lax.axis_index("s")
    wid = sid * NUM_CORES + cid
    base = wid * PER_W
    pltpu.sync_copy(idx_hbm.at[wid], idx_v)
    gcp = [None] * NBUF
    wcp = [None] * NBUF
    for p in range(2):
        gcp[p] = pltpu.async_copy(table_hbm.at[idx_v.at[p]], bufs[p], gsems[p])
    for c in range(NCHUNK):
        b = c % NBUF
        gcp[b].wait()
        wcp[b] = pltpu.async_copy(
            bufs[b], out_hbm.at[pl.ds(base + c * CHUNK, CHUNK)], wsems[b]
        )
        g = c + 2
        if g < NCHUNK:
            gb = g % NBUF
            if g >= NBUF:
                wcp[gb].wait()
            gcp[gb] = pltpu.async_copy(
                table_hbm.at[idx_v.at[g]], bufs[gb], gsems[gb]
            )
    for c in range(NCHUNK - NBUF, NCHUNK):
        wcp[c % NBUF].wait()


@jax.jit
def kernel(position_ids, embeddings_table):
    idx = position_ids.reshape(NW, NCHUNK, CHUNK)
    out = pl.kernel(
        _gather_body,
        out_type=jax.ShapeDtypeStruct((N, EMB), jnp.float32),
        mesh=plsc.VectorSubcoreMesh(core_axis_name="c", subcore_axis_name="s"),
        scratch_types=(
            [pltpu.VMEM((NCHUNK, CHUNK), jnp.int32)]
            + [pltpu.VMEM((CHUNK, EMB), jnp.float32)] * NBUF
            + [pltpu.SemaphoreType.DMA] * (2 * NBUF)
        ),
    )(idx, embeddings_table)
    return out.reshape(BATCH, SEQ_LEN, EMB)
